# baseline (speedup 1.0000x reference)
import functools

import jax
import jax.numpy as jnp
from jax import lax
from jax.experimental import pallas as pl
from jax.experimental.pallas import tpu as pltpu

_EPS = 1e-5


def _fold(gamma, beta, mean, var):
    s = gamma / jnp.sqrt(var + _EPS)
    return s, beta - mean * s


def _block_kernel(x_ref, w1_ref, b1_ref, w2_ref, b2_ref, o_ref,
                  xs_ref, ys_ref, *, H, W, C, B):
    HW = H * W
    lead = W + 1
    data0 = lead
    data1 = lead + HW

    col = lax.broadcasted_iota(jnp.int32, (1, HW), 1) % W
    m_w0 = (col != 0).astype(jnp.bfloat16)
    m_wl = (col != W - 1).astype(jnp.bfloat16)

    def stage(slabs_ref, b, data):
        z = jnp.zeros((C, lead), jnp.bfloat16)
        for v, d in ((0, data), (1, data * m_wl), (2, data * m_w0)):
            slabs_ref[b, v, :, pl.ds(0, data0)] = z
            slabs_ref[b, v, :, pl.ds(data0, HW)] = d
            slabs_ref[b, v, :, pl.ds(data1, lead)] = z

    _V = (1, 0, 2)

    def conv_cols(slabs_ref, b):
        taps = []
        for kh in range(3):
            for kw in range(3):
                taps.append(slabs_ref[b, _V[kw], :, pl.ds(kh * W + kw, HW)])
        return jnp.concatenate(taps, axis=0)

    for b in range(B):
        x = x_ref[b]
        stage(xs_ref, b, x)
        y1 = jnp.dot(w1_ref[...], conv_cols(xs_ref, b),
                     preferred_element_type=jnp.float32)
        y1 = jnp.maximum(y1 + b1_ref[...], 0.0).astype(jnp.bfloat16)
        stage(ys_ref, b, y1)
        y2 = jnp.dot(w2_ref[...], conv_cols(ys_ref, b),
                     preferred_element_type=jnp.float32)
        o_ref[b] = y2 + b2_ref[...] + x.astype(jnp.float32)


@jax.jit
def _basic_block(x, w1, g1, b1, m1, v1, w2, g2, b2, m2, v2):
    N, C, H, W = x.shape
    HW = H * W
    slab = HW + 2 * (W + 1)
    B = 8 if N % 8 == 0 else 1

    s1, bb1 = _fold(g1, b1, m1, v1)
    s2, bb2 = _fold(g2, b2, m2, v2)
    w1c = (w1 * s1).reshape(9, C, C).transpose(2, 0, 1)
    w1c = w1c.reshape(C, 9 * C).astype(jnp.bfloat16)
    w2c = (w2 * s2).reshape(9, C, C).transpose(2, 0, 1)
    w2c = w2c.reshape(C, 9 * C).astype(jnp.bfloat16)
    bb1 = bb1.reshape(C, 1).astype(jnp.float32)
    bb2 = bb2.reshape(C, 1).astype(jnp.float32)

    xb = x.reshape(N, C, HW).astype(jnp.bfloat16)

    kern = functools.partial(_block_kernel, H=H, W=W, C=C, B=B)
    out = pl.pallas_call(
        kern,
        out_shape=jax.ShapeDtypeStruct((N, C, HW), jnp.float32),
        grid=(N // B,),
        in_specs=[
            pl.BlockSpec((B, C, HW), lambda n: (n, 0, 0)),
            pl.BlockSpec((C, 9 * C), lambda n: (0, 0)),
            pl.BlockSpec((C, 1), lambda n: (0, 0)),
            pl.BlockSpec((C, 9 * C), lambda n: (0, 0)),
            pl.BlockSpec((C, 1), lambda n: (0, 0)),
        ],
        out_specs=pl.BlockSpec((B, C, HW), lambda n: (n, 0, 0)),
        scratch_shapes=[
            pltpu.VMEM((B, 3, C, slab), jnp.bfloat16),
            pltpu.VMEM((B, 3, C, slab), jnp.bfloat16),
        ],
        compiler_params=pltpu.CompilerParams(
            dimension_semantics=("arbitrary",)),
    )(xb, w1c, bb1, w2c, bb2)
    return out.reshape(N, C, H, W)


def kernel(x, w1, g1, b1, m1, v1, w2, g2, b2, m2, v2,
           wds, bds, gds, bds_bn, mds, vds):
    del wds, bds, gds, bds_bn, mds, vds
    return _basic_block(x, w1, g1, b1, m1, v1, w2, g2, b2, m2, v2)


# speedup vs baseline: 1.0654x; 1.0654x over previous
import functools

import jax
import jax.numpy as jnp
from jax import lax
from jax.experimental import pallas as pl
from jax.experimental.pallas import tpu as pltpu

_EPS = 1e-5


def _fold(gamma, beta, mean, var):
    s = gamma / jnp.sqrt(var + _EPS)
    return s, beta - mean * s


def _block_kernel(x_ref, w1_ref, b1_ref, w2_ref, b2_ref, o_ref,
                  xp_ref, y1p_ref, *, H, W, C):
    HW = H * W
    lead = W + 1
    data0 = lead
    data1 = lead + HW

    col = lax.broadcasted_iota(jnp.int32, (1, HW), 1) % W
    mask_l = col != 0
    mask_r = col != W - 1

    def cols_from(slab_ref):
        taps = []
        for kh in range(3):
            for kw in range(3):
                t = slab_ref[:, pl.ds(kh * W + kw, HW)]
                if kw == 0:
                    t = jnp.where(mask_l, t, jnp.bfloat16(0))
                elif kw == 2:
                    t = jnp.where(mask_r, t, jnp.bfloat16(0))
                taps.append(t)
        return jnp.concatenate(taps, axis=0)

    x = x_ref[0]
    xp_ref[:, pl.ds(0, data0)] = jnp.zeros((C, data0), jnp.bfloat16)
    xp_ref[:, pl.ds(data0, HW)] = x.astype(jnp.bfloat16)
    xp_ref[:, pl.ds(data1, lead)] = jnp.zeros((C, lead), jnp.bfloat16)

    y1 = jnp.dot(w1_ref[...], cols_from(xp_ref),
                 preferred_element_type=jnp.float32)
    y1 = jnp.maximum(y1 + b1_ref[...], 0.0).astype(jnp.bfloat16)

    y1p_ref[:, pl.ds(0, data0)] = jnp.zeros((C, data0), jnp.bfloat16)
    y1p_ref[:, pl.ds(data0, HW)] = y1
    y1p_ref[:, pl.ds(data1, lead)] = jnp.zeros((C, lead), jnp.bfloat16)

    y2 = jnp.dot(w2_ref[...], cols_from(y1p_ref),
                 preferred_element_type=jnp.float32)
    o_ref[0] = y2 + b2_ref[...] + x


@jax.jit
def _basic_block(x, w1, g1, b1, m1, v1, w2, g2, b2, m2, v2):
    N, C, H, W = x.shape
    HW = H * W
    slab = HW + 2 * (W + 1)

    s1, bb1 = _fold(g1, b1, m1, v1)
    s2, bb2 = _fold(g2, b2, m2, v2)
    w1c = (w1 * s1).reshape(9, C, C).transpose(2, 0, 1)
    w1c = w1c.reshape(C, 9 * C).astype(jnp.bfloat16)
    w2c = (w2 * s2).reshape(9, C, C).transpose(2, 0, 1)
    w2c = w2c.reshape(C, 9 * C).astype(jnp.bfloat16)
    bb1 = bb1.reshape(C, 1).astype(jnp.float32)
    bb2 = bb2.reshape(C, 1).astype(jnp.float32)

    kern = functools.partial(_block_kernel, H=H, W=W, C=C)
    out = pl.pallas_call(
        kern,
        out_shape=jax.ShapeDtypeStruct((N, C, HW), jnp.float32),
        grid=(N,),
        in_specs=[
            pl.BlockSpec((1, C, HW), lambda n: (n, 0, 0)),
            pl.BlockSpec((C, 9 * C), lambda n: (0, 0)),
            pl.BlockSpec((C, 1), lambda n: (0, 0)),
            pl.BlockSpec((C, 9 * C), lambda n: (0, 0)),
            pl.BlockSpec((C, 1), lambda n: (0, 0)),
        ],
        out_specs=pl.BlockSpec((1, C, HW), lambda n: (n, 0, 0)),
        scratch_shapes=[
            pltpu.VMEM((C, slab), jnp.bfloat16),
            pltpu.VMEM((C, slab), jnp.bfloat16),
        ],
        compiler_params=pltpu.CompilerParams(
            dimension_semantics=("parallel",),
            allow_input_fusion=[True, False, False, False, False]),
    )(x.reshape(N, C, HW), w1c, bb1, w2c, bb2)
    return out.reshape(N, C, H, W)


def kernel(x, w1, g1, b1, m1, v1, w2, g2, b2, m2, v2,
           wds, bds, gds, bds_bn, mds, vds):
    del wds, bds, gds, bds_bn, mds, vds
    return _basic_block(x, w1, g1, b1, m1, v1, w2, g2, b2, m2, v2)


# speedup vs baseline: 1.0965x; 1.0292x over previous
import functools

import jax
import jax.numpy as jnp
from jax import lax
from jax.experimental import pallas as pl
from jax.experimental.pallas import tpu as pltpu

_EPS = 1e-5


def _fold(gamma, beta, mean, var):
    s = gamma / jnp.sqrt(var + _EPS)
    return s, beta - mean * s


def _block_kernel(x_ref, w1_ref, b1_ref, w2_ref, b2_ref, o_ref,
                  xp_ref, y1p_ref, *, H, W, C, B):
    HW = H * W
    lead = W + 1
    data0 = lead
    data1 = lead + HW

    col = lax.broadcasted_iota(jnp.int32, (1, HW), 1) % W
    mask_l = (col != 0).astype(jnp.bfloat16)
    mask_r = (col != W - 1).astype(jnp.bfloat16)

    def cols_from(slab_ref, b):
        taps = []
        for kh in range(3):
            for kw in range(3):
                t = slab_ref[b, :, pl.ds(kh * W + kw, HW)]
                if kw == 0:
                    t = t * mask_l
                elif kw == 2:
                    t = t * mask_r
                taps.append(t)
        return jnp.concatenate(taps, axis=0)

    def stage(slab_ref, b, data):
        slab_ref[b, :, pl.ds(0, data0)] = jnp.zeros((C, data0), jnp.bfloat16)
        slab_ref[b, :, pl.ds(data0, HW)] = data
        slab_ref[b, :, pl.ds(data1, lead)] = jnp.zeros((C, lead), jnp.bfloat16)

    for b in range(B):
        x = x_ref[b]
        stage(xp_ref, b, x.astype(jnp.bfloat16))
        y1 = jnp.dot(w1_ref[...], cols_from(xp_ref, b),
                     preferred_element_type=jnp.float32)
        y1 = jnp.maximum(y1 + b1_ref[...], 0.0).astype(jnp.bfloat16)
        stage(y1p_ref, b, y1)
        y2 = jnp.dot(w2_ref[...], cols_from(y1p_ref, b),
                     preferred_element_type=jnp.float32)
        o_ref[b] = y2 + b2_ref[...] + x


@jax.jit
def _basic_block(x, w1, g1, b1, m1, v1, w2, g2, b2, m2, v2):
    N, C, H, W = x.shape
    HW = H * W
    slab = HW + 2 * (W + 1)

    s1, bb1 = _fold(g1, b1, m1, v1)
    s2, bb2 = _fold(g2, b2, m2, v2)
    w1c = (w1 * s1).reshape(9, C, C).transpose(2, 0, 1)
    w1c = w1c.reshape(C, 9 * C).astype(jnp.bfloat16)
    w2c = (w2 * s2).reshape(9, C, C).transpose(2, 0, 1)
    w2c = w2c.reshape(C, 9 * C).astype(jnp.bfloat16)
    bb1 = bb1.reshape(C, 1).astype(jnp.float32)
    bb2 = bb2.reshape(C, 1).astype(jnp.float32)

    B = 2 if N % 2 == 0 else 1
    kern = functools.partial(_block_kernel, H=H, W=W, C=C, B=B)
    out = pl.pallas_call(
        kern,
        out_shape=jax.ShapeDtypeStruct((N, C, HW), jnp.float32),
        grid=(N // B,),
        in_specs=[
            pl.BlockSpec((B, C, HW), lambda n: (n, 0, 0)),
            pl.BlockSpec((C, 9 * C), lambda n: (0, 0)),
            pl.BlockSpec((C, 1), lambda n: (0, 0)),
            pl.BlockSpec((C, 9 * C), lambda n: (0, 0)),
            pl.BlockSpec((C, 1), lambda n: (0, 0)),
        ],
        out_specs=pl.BlockSpec((B, C, HW), lambda n: (n, 0, 0)),
        scratch_shapes=[
            pltpu.VMEM((B, C, slab), jnp.bfloat16),
            pltpu.VMEM((B, C, slab), jnp.bfloat16),
        ],
        compiler_params=pltpu.CompilerParams(
            dimension_semantics=("parallel",),
            allow_input_fusion=[True, False, False, False, False]),
    )(x.reshape(N, C, HW), w1c, bb1, w2c, bb2)
    return out.reshape(N, C, H, W)


def kernel(x, w1, g1, b1, m1, v1, w2, g2, b2, m2, v2,
           wds, bds, gds, bds_bn, mds, vds):
    del wds, bds, gds, bds_bn, mds, vds
    return _basic_block(x, w1, g1, b1, m1, v1, w2, g2, b2, m2, v2)


# speedup vs baseline: 1.1062x; 1.0088x over previous
import functools

import jax
import jax.numpy as jnp
from jax import lax
from jax.experimental import pallas as pl
from jax.experimental.pallas import tpu as pltpu

_EPS = 1e-5


def _fold(gamma, beta, mean, var):
    s = gamma / jnp.sqrt(var + _EPS)
    return s, beta - mean * s


def _block_kernel(x_ref, w1_ref, b1_ref, w2_ref, b2_ref, o_ref,
                  xp_ref, y1p_ref, *, H, W, C, B):
    HW = H * W
    lead = W + 1
    data0 = lead
    data1 = lead + HW

    col = lax.broadcasted_iota(jnp.int32, (1, HW), 1) % W
    mask_l = (col != 0).astype(jnp.bfloat16)
    mask_r = (col != W - 1).astype(jnp.bfloat16)

    def cols_from(slab_ref, b):
        taps = []
        for kh in range(3):
            for kw in range(3):
                t = slab_ref[b, :, pl.ds(kh * W + kw, HW)]
                if kw == 0:
                    t = t * mask_l
                elif kw == 2:
                    t = t * mask_r
                taps.append(t)
        return jnp.concatenate(taps, axis=0)

    def stage(slab_ref, b, data):
        slab_ref[b, :, pl.ds(0, data0)] = jnp.zeros((C, data0), jnp.bfloat16)
        slab_ref[b, :, pl.ds(data0, HW)] = data
        slab_ref[b, :, pl.ds(data1, lead)] = jnp.zeros((C, lead), jnp.bfloat16)

    for b in range(B):
        x = x_ref[b]
        stage(xp_ref, b, x.astype(jnp.bfloat16))
        y1 = jnp.dot(w1_ref[...], cols_from(xp_ref, b),
                     preferred_element_type=jnp.float32)
        y1 = jnp.maximum(y1 + b1_ref[...], 0.0).astype(jnp.bfloat16)
        stage(y1p_ref, b, y1)
        y2 = jnp.dot(w2_ref[...], cols_from(y1p_ref, b),
                     preferred_element_type=jnp.float32)
        o_ref[b] = y2 + b2_ref[...] + x


@jax.jit
def _basic_block(x, w1, g1, b1, m1, v1, w2, g2, b2, m2, v2):
    N, C, H, W = x.shape
    HW = H * W
    slab = HW + 2 * (W + 1)

    s1, bb1 = _fold(g1, b1, m1, v1)
    s2, bb2 = _fold(g2, b2, m2, v2)
    w1c = (w1 * s1).reshape(9, C, C).transpose(2, 0, 1)
    w1c = w1c.reshape(C, 9 * C).astype(jnp.bfloat16)
    w2c = (w2 * s2).reshape(9, C, C).transpose(2, 0, 1)
    w2c = w2c.reshape(C, 9 * C).astype(jnp.bfloat16)
    bb1 = bb1.reshape(C, 1).astype(jnp.float32)
    bb2 = bb2.reshape(C, 1).astype(jnp.float32)

    B = 4 if N % 4 == 0 else 1
    kern = functools.partial(_block_kernel, H=H, W=W, C=C, B=B)
    out = pl.pallas_call(
        kern,
        out_shape=jax.ShapeDtypeStruct((N, C, HW), jnp.float32),
        grid=(N // B,),
        in_specs=[
            pl.BlockSpec((B, C, HW), lambda n: (n, 0, 0)),
            pl.BlockSpec((C, 9 * C), lambda n: (0, 0)),
            pl.BlockSpec((C, 1), lambda n: (0, 0)),
            pl.BlockSpec((C, 9 * C), lambda n: (0, 0)),
            pl.BlockSpec((C, 1), lambda n: (0, 0)),
        ],
        out_specs=pl.BlockSpec((B, C, HW), lambda n: (n, 0, 0)),
        scratch_shapes=[
            pltpu.VMEM((B, C, slab), jnp.bfloat16),
            pltpu.VMEM((B, C, slab), jnp.bfloat16),
        ],
        compiler_params=pltpu.CompilerParams(
            dimension_semantics=("parallel",),
            allow_input_fusion=[True, False, False, False, False]),
    )(x.reshape(N, C, HW), w1c, bb1, w2c, bb2)
    return out.reshape(N, C, H, W)


def kernel(x, w1, g1, b1, m1, v1, w2, g2, b2, m2, v2,
           wds, bds, gds, bds_bn, mds, vds):
    del wds, bds, gds, bds_bn, mds, vds
    return _basic_block(x, w1, g1, b1, m1, v1, w2, g2, b2, m2, v2)
